# Initial kernel scaffold; baseline (speedup 1.0000x reference)
import jax
import jax.numpy as jnp
from jax.experimental import pallas as pl


def kernel(token_ids, weights):
    raise NotImplementedError("write your pallas kernel here")


# speedup vs baseline: 1.6839x; 1.6839x over previous
import functools

import jax
import jax.numpy as jnp
from jax import lax
from jax.experimental import pallas as pl
from jax.experimental.pallas import tpu as pltpu
from jax.experimental.pallas import tpu_sc as plsc

_NUM_CORES = 2
_NUM_SUBCORES = 16
_NUM_WORKERS = _NUM_CORES * _NUM_SUBCORES
_CHUNK = 128


@functools.lru_cache(maxsize=None)
def _make_gather(n_chunks: int, dim: int):
    mesh = plsc.VectorSubcoreMesh(core_axis_name="c", subcore_axis_name="s")

    @functools.partial(
        pl.kernel,
        mesh=mesh,
        out_type=jax.ShapeDtypeStruct(
            (_NUM_WORKERS, n_chunks, _CHUNK, dim), jnp.float32
        ),
        scratch_types=[
            pltpu.VMEM((n_chunks, _CHUNK), jnp.int32),
            pltpu.VMEM((_CHUNK, dim), jnp.float32),
            pltpu.SemaphoreType.DMA,
        ],
        compiler_params=pltpu.CompilerParams(use_tc_tiling_on_sc=False),
    )
    def gather_kernel(table_hbm, idx_hbm, out_hbm, idx_v, rows_v, sem):
        wid = lax.axis_index("s") * _NUM_CORES + lax.axis_index("c")
        pltpu.sync_copy(idx_hbm.at[wid], idx_v)

        def body(j, carry):
            pltpu.async_copy(table_hbm.at[idx_v.at[j]], rows_v, sem).wait()
            pltpu.sync_copy(rows_v, out_hbm.at[wid, j])
            return carry

        lax.fori_loop(0, n_chunks, body, 0)

    return gather_kernel


def kernel(token_ids, weights):
    n_tok, n_seq = token_ids.shape
    dim = weights.shape[1]
    total = n_tok * n_seq
    assert total % (_NUM_WORKERS * _CHUNK) == 0
    n_chunks = total // (_NUM_WORKERS * _CHUNK)
    idx = token_ids.reshape(_NUM_WORKERS, n_chunks, _CHUNK).astype(jnp.int32)
    out = _make_gather(n_chunks, dim)(weights, idx)
    return out.reshape(n_tok, n_seq, dim)


# speedup vs baseline: 1.8690x; 1.1099x over previous
import functools

import jax
import jax.numpy as jnp
from jax import lax
from jax.experimental import pallas as pl
from jax.experimental.pallas import tpu as pltpu
from jax.experimental.pallas import tpu_sc as plsc

_NUM_CORES = 2
_NUM_SUBCORES = 16
_NUM_WORKERS = _NUM_CORES * _NUM_SUBCORES
_CHUNK = 128
_SUPER = 4


@functools.lru_cache(maxsize=None)
def _make_gather(n_super: int, dim: int):
    assert n_super >= 2 and n_super % 2 == 0
    n_chunks = n_super * _SUPER
    rows = _SUPER * _CHUNK
    mesh = plsc.VectorSubcoreMesh(core_axis_name="c", subcore_axis_name="s")

    @functools.partial(
        pl.kernel,
        mesh=mesh,
        out_type=jax.ShapeDtypeStruct(
            (_NUM_WORKERS, n_super, rows, dim), jnp.float32
        ),
        scratch_types=[
            pltpu.VMEM((n_chunks, _CHUNK), jnp.int32),
            pltpu.VMEM((rows, dim), jnp.float32),
            pltpu.VMEM((rows, dim), jnp.float32),
            pltpu.SemaphoreType.DMA,
            pltpu.SemaphoreType.DMA,
            pltpu.SemaphoreType.DMA,
            pltpu.SemaphoreType.DMA,
        ],
        compiler_params=pltpu.CompilerParams(use_tc_tiling_on_sc=False),
    )
    def gather_kernel(table_hbm, idx_hbm, out_hbm, idx_v, buf0, buf1,
                      gs0, gs1, ss0, ss1):
        wid = lax.axis_index("s") * _NUM_CORES + lax.axis_index("c")
        pltpu.sync_copy(idx_hbm.at[wid], idx_v)
        bufs = (buf0, buf1)
        gsems = (gs0, gs1)
        ssems = (ss0, ss1)

        def fire_gathers(s, p):
            for c in range(_SUPER):
                pltpu.async_copy(
                    table_hbm.at[idx_v.at[s * _SUPER + c]],
                    bufs[p].at[pl.ds(c * _CHUNK, _CHUNK)],
                    gsems[p],
                )

        def drain_gathers(p):
            pltpu.make_async_copy(
                table_hbm.at[pl.ds(0, rows)], bufs[p], gsems[p]
            ).wait()

        def fire_store(s, p):
            pltpu.async_copy(bufs[p], out_hbm.at[wid, s], ssems[p])

        def wait_store(p):
            pltpu.make_async_copy(
                bufs[p], out_hbm.at[wid, 0], ssems[p]
            ).wait()

        fire_gathers(0, 0)
        drain_gathers(0)
        fire_gathers(1, 1)
        fire_store(0, 0)

        def pair(t, carry):
            s = 2 * t + 1
            drain_gathers(1)
            wait_store(0)
            fire_gathers(s + 1, 0)
            fire_store(s, 1)
            drain_gathers(0)
            wait_store(1)
            fire_gathers(s + 2, 1)
            fire_store(s + 1, 0)
            return carry

        lax.fori_loop(0, (n_super - 2) // 2, pair, 0)

        drain_gathers(1)
        fire_store(n_super - 1, 1)
        wait_store(0)
        wait_store(1)

    return gather_kernel


def kernel(token_ids, weights):
    n_tok, n_seq = token_ids.shape
    dim = weights.shape[1]
    total = n_tok * n_seq
    assert total % (_NUM_WORKERS * _SUPER * _CHUNK) == 0
    n_super = total // (_NUM_WORKERS * _SUPER * _CHUNK)
    idx = token_ids.reshape(
        _NUM_WORKERS, n_super * _SUPER, _CHUNK
    ).astype(jnp.int32)
    out = _make_gather(n_super, dim)(weights, idx)
    return out.reshape(n_tok, n_seq, dim)
